# baseline (speedup 1.0000x reference)
import functools

import jax
import jax.numpy as jnp
from jax import lax
from jax.experimental import pallas as pl
from jax.experimental.pallas import tpu as pltpu
from jax.experimental.pallas import tpu_sc as plsc

B_, L, H = 1, 2048, 2048
NH, DH = 16, 128
BS = 32
NB = L // BS
NSEL = 16
WIN = 256
QT = 256
NQT = L // QT
SCALE = 1.0 / (DH ** 0.5)
NEG = -1e9


def _proj_body_hi(a_ref, b_ref, bias_ref, o_ref):
    acc = lax.dot_general(a_ref[...], b_ref[...], (((1,), (0,)), ((), ())),
                          preferred_element_type=jnp.float32)
    o_ref[...] = acc + bias_ref[0:1, :]


def _proj_body(a_ref, b_ref, bias_ref, o_ref):
    acc = lax.dot_general(a_ref[...].astype(jnp.bfloat16),
                          b_ref[...].astype(jnp.bfloat16),
                          (((1,), (0,)), ((), ())),
                          preferred_element_type=jnp.float32)
    o_ref[...] = acc + bias_ref[0:1, :]


def _projection(hs, wall, ball):
    nw = wall.shape[1] // 128
    nj = 7 if nw % 7 == 0 else nw
    bn = (nw // nj) * 128
    bm = 512
    return pl.pallas_call(
        _proj_body_hi,
        grid=(L // bm, nj),
        in_specs=[
            pl.BlockSpec((bm, H), lambda i, j: (i, 0)),
            pl.BlockSpec((H, bn), lambda i, j: (0, j)),
            pl.BlockSpec((8, bn), lambda i, j: (0, j)),
        ],
        out_specs=pl.BlockSpec((bm, bn), lambda i, j: (i, j)),
        out_shape=jax.ShapeDtypeStruct((L, nw * 128), jnp.float32),
        compiler_params=pltpu.CompilerParams(
            dimension_semantics=("parallel", "parallel")),
    )(hs, wall, jnp.broadcast_to(ball, (8, nw * 128)))


def _compress_body(q_ref, k_ref, v_ref, comp_ref, idx_ref, kh_ref, vh_ref):
    h = pl.program_id(0)
    q = q_ref[...]
    k = k_ref[...]
    v = v_ref[...]
    kh_ref[0] = k
    vh_ref[0] = v

    ck = jnp.mean(k.reshape(NB, BS, DH), axis=1)
    cv = jnp.mean(v.reshape(NB, BS, DH), axis=1)

    s = lax.dot_general(q, ck, (((1,), (1,)), ((), ())),
                        preferred_element_type=jnp.float32) * SCALE
    m = jnp.max(s, axis=-1, keepdims=True)
    e = jnp.exp(s - m)
    p = e / jnp.sum(e, axis=-1, keepdims=True)
    comp_ref[...] = lax.dot_general(p, cv, (((1,), (0,)), ((), ())),
                                    preferred_element_type=jnp.float32)

    imp = jnp.sum(p, axis=0, keepdims=True)
    eye = jnp.where(lax.broadcasted_iota(jnp.int32, (NB, NB), 0)
                    == lax.broadcasted_iota(jnp.int32, (NB, NB), 1),
                    jnp.float32(1.0), jnp.float32(0.0))
    imp_c = lax.dot_general(eye, imp, (((1,), (1,)), ((), ())),
                            preferred_element_type=jnp.float32)

    gt = imp > imp_c
    tie = (imp == imp_c) & (lax.broadcasted_iota(jnp.int32, (NB, NB), 1)
                            < lax.broadcasted_iota(jnp.int32, (NB, NB), 0))
    rank = jnp.sum((gt | tie).astype(jnp.float32), axis=1, keepdims=True)
    selm = rank < NSEL

    slot_i = lax.broadcasted_iota(jnp.int32, (NB, NSEL), 1).astype(jnp.float32)
    oh = ((rank == slot_i) & selm).astype(jnp.float32)
    r_i = lax.broadcasted_iota(jnp.int32, (NB, NSEL), 0).astype(jnp.float32)
    blk_row = jnp.sum(r_i * oh, axis=0, keepdims=True)

    pos = lax.broadcasted_iota(jnp.int32, (NSEL * BS, 1), 0)
    slot_of = pos // BS
    oh_pos = (lax.broadcasted_iota(jnp.int32, (NSEL * BS, NSEL), 1)
              == slot_of).astype(jnp.float32)
    blk_of = jnp.sum(oh_pos * blk_row, axis=1, keepdims=True)
    idx_ref[0] = (blk_of.astype(jnp.int32) * BS
                  + (pos - slot_of * BS) + h * L)


def _compress_select(y):
    return pl.pallas_call(
        _compress_body,
        grid=(NH,),
        in_specs=[
            pl.BlockSpec((L, DH), lambda h: (0, h)),
            pl.BlockSpec((L, DH), lambda h: (0, NH + h)),
            pl.BlockSpec((L, DH), lambda h: (0, 2 * NH + h)),
        ],
        out_specs=[
            pl.BlockSpec((L, DH), lambda h: (0, h)),
            pl.BlockSpec((1, NSEL * BS, 1), lambda h: (h, 0, 0)),
            pl.BlockSpec((1, L, DH), lambda h: (h, 0, 0)),
            pl.BlockSpec((1, L, DH), lambda h: (h, 0, 0)),
        ],
        out_shape=[
            jax.ShapeDtypeStruct((L, H), jnp.float32),
            jax.ShapeDtypeStruct((NH, NSEL * BS, 1), jnp.int32),
            jax.ShapeDtypeStruct((NH, L, DH), jnp.float32),
            jax.ShapeDtypeStruct((NH, L, DH), jnp.float32),
        ],
        compiler_params=pltpu.CompilerParams(
            dimension_semantics=("parallel",)),
    )(y, y, y)


NROW = NSEL * BS
NCH = NROW // 128


def _sc_gather_body(ktab, vtab, idx_hbm, out_hbm, idx_v, rows_v, sem):
    c = lax.axis_index("c")
    s = lax.axis_index("s")
    pltpu.sync_copy(idx_hbm.at[s], idx_v)

    @pl.when(c == 0)
    def _():
        cps = [pltpu.async_copy(ktab.at[idx_v.at[j]],
                                rows_v.at[pl.ds(j * 128, 128)], sem)
               for j in range(NCH)]
        for cp in cps:
            cp.wait()

    @pl.when(c == 1)
    def _():
        cps = [pltpu.async_copy(vtab.at[idx_v.at[j]],
                                rows_v.at[pl.ds(j * 128, 128)], sem)
               for j in range(NCH)]
        for cp in cps:
            cp.wait()

    pltpu.sync_copy(rows_v, out_hbm.at[c * NH + s])


def _sc_gather(ktab, vtab, idx3):
    mesh = plsc.VectorSubcoreMesh(core_axis_name="c", subcore_axis_name="s")
    fn = pl.kernel(
        _sc_gather_body,
        out_type=jax.ShapeDtypeStruct((2 * NH, NROW, DH), jnp.float32),
        mesh=mesh,
        scratch_types=[
            pltpu.VMEM((NCH, 128), jnp.int32),
            pltpu.VMEM((NROW, DH), jnp.float32),
            pltpu.SemaphoreType.DMA,
        ],
    )
    return fn(ktab, vtab, idx3)


def _attn_body(q_ref, kc_ref, kp_ref, vc_ref, vp_ref, sk_ref, sv_ref,
               comp_ref, gate_ref, o_ref):
    qi = pl.program_id(1)
    q = q_ref[...].astype(jnp.bfloat16)

    sk = sk_ref[0].astype(jnp.bfloat16)
    sv = sv_ref[0].astype(jnp.bfloat16)
    ss = lax.dot_general(q, sk, (((1,), (1,)), ((), ())),
                         preferred_element_type=jnp.float32) * SCALE
    ms = jnp.max(ss, axis=-1, keepdims=True)
    es = jnp.exp(ss - ms)
    sel_out = lax.dot_general(es.astype(jnp.bfloat16), sv,
                              (((1,), (0,)), ((), ())),
                              preferred_element_type=jnp.float32)
    sel_out = sel_out / jnp.sum(es, axis=-1, keepdims=True)

    i_ = lax.broadcasted_iota(jnp.int32, (QT, QT), 0)
    j_ = lax.broadcasted_iota(jnp.int32, (QT, QT), 1)
    kc = kc_ref[0].astype(jnp.bfloat16)
    vc = vc_ref[0].astype(jnp.bfloat16)
    kp = kp_ref[0].astype(jnp.bfloat16)
    vp = vp_ref[0].astype(jnp.bfloat16)
    sc = lax.dot_general(q, kc, (((1,), (1,)), ((), ())),
                         preferred_element_type=jnp.float32) * SCALE
    sc = jnp.where(i_ >= j_, sc, NEG)
    sp = lax.dot_general(q, kp, (((1,), (1,)), ((), ())),
                         preferred_element_type=jnp.float32) * SCALE
    sp = jnp.where((j_ > i_) & (qi > 0), sp, NEG)
    m = jnp.maximum(jnp.max(sc, axis=-1, keepdims=True),
                    jnp.max(sp, axis=-1, keepdims=True))
    ec = jnp.exp(sc - m)
    ep = jnp.exp(sp - m)
    den = jnp.sum(ec, axis=-1, keepdims=True) + jnp.sum(ep, axis=-1,
                                                        keepdims=True)
    sl_out = (lax.dot_general(ec.astype(jnp.bfloat16), vc,
                              (((1,), (0,)), ((), ())),
                              preferred_element_type=jnp.float32)
              + lax.dot_general(ep.astype(jnp.bfloat16), vp,
                                (((1,), (0,)), ((), ())),
                                preferred_element_type=jnp.float32)) / den

    g = jax.nn.sigmoid(gate_ref[...])
    o_ref[...] = (g[:, 0:1] * comp_ref[...]
                  + g[:, 1:2] * sel_out
                  + g[:, 2:3] * sl_out)


def _attend_combine(y, khead, vhead, skv, comp):
    return pl.pallas_call(
        _attn_body,
        grid=(NH, NQT),
        in_specs=[
            pl.BlockSpec((QT, DH), lambda h, qi: (qi, h)),
            pl.BlockSpec((1, QT, DH), lambda h, qi: (h, qi, 0)),
            pl.BlockSpec((1, QT, DH),
                         lambda h, qi: (h, jnp.maximum(qi - 1, 0), 0)),
            pl.BlockSpec((1, QT, DH), lambda h, qi: (h, qi, 0)),
            pl.BlockSpec((1, QT, DH),
                         lambda h, qi: (h, jnp.maximum(qi - 1, 0), 0)),
            pl.BlockSpec((1, NROW, DH), lambda h, qi: (h, 0, 0)),
            pl.BlockSpec((1, NROW, DH), lambda h, qi: (NH + h, 0, 0)),
            pl.BlockSpec((QT, DH), lambda h, qi: (qi, h)),
            pl.BlockSpec((QT, 128), lambda h, qi: (qi, 3 * NH)),
        ],
        out_specs=pl.BlockSpec((QT, DH), lambda h, qi: (qi, h)),
        out_shape=jax.ShapeDtypeStruct((L, H), jnp.float32),
        compiler_params=pltpu.CompilerParams(
            dimension_semantics=("parallel", "arbitrary")),
    )(y, khead, khead, vhead, vhead, skv, skv, comp, y)


def _out_proj(attn, wo, bo):
    return pl.pallas_call(
        _proj_body,
        grid=(L // 512, H // 512),
        in_specs=[
            pl.BlockSpec((512, H), lambda i, j: (i, 0)),
            pl.BlockSpec((H, 512), lambda i, j: (0, j)),
            pl.BlockSpec((8, 512), lambda i, j: (0, j)),
        ],
        out_specs=pl.BlockSpec((512, 512), lambda i, j: (i, j)),
        out_shape=jax.ShapeDtypeStruct((L, H), jnp.float32),
        compiler_params=pltpu.CompilerParams(
            dimension_semantics=("parallel", "parallel")),
    )(attn, wo, jnp.broadcast_to(bo, (8, H)))


@jax.jit
def kernel(hidden_states, Wq, bq, Wk, bk, Wv, bv, Wo, bo, Wg, bg):
    hs = hidden_states.reshape(L, H)
    wg_pad = jnp.zeros((H, 128), jnp.float32).at[:, :3].set(Wg)
    bg_pad = jnp.zeros((128,), jnp.float32).at[:3].set(bg)
    wall = jnp.concatenate([Wq, Wk, Wv, wg_pad], axis=1)
    ball = jnp.concatenate([bq, bk, bv, bg_pad])

    y = _projection(hs, wall, ball)
    comp, idx, khead, vhead = _compress_select(y)
    skv = _sc_gather(khead.reshape(NH * L, DH),
                     vhead.reshape(NH * L, DH),
                     idx.reshape(NH, NCH, 128))
    attn = _attend_combine(y, khead, vhead, skv, comp)
    out = _out_proj(attn, Wo, bo)
    return out.reshape(B_, L, H)


# speedup vs baseline: 1.1641x; 1.1641x over previous
import functools

import jax
import jax.numpy as jnp
from jax import lax
from jax.experimental import pallas as pl
from jax.experimental.pallas import tpu as pltpu
from jax.experimental.pallas import tpu_sc as plsc

B_, L, H = 1, 2048, 2048
NH, DH = 16, 128
BS = 32
NB = L // BS
NSEL = 16
WIN = 256
QT = 256
NQT = L // QT
SCALE = 1.0 / (DH ** 0.5)
NEG = -1e9


def _proj_body_hi(a_ref, b_ref, bias_ref, o_ref):
    acc = lax.dot_general(a_ref[...], b_ref[...], (((1,), (0,)), ((), ())),
                          preferred_element_type=jnp.float32)
    o_ref[...] = acc + bias_ref[0:1, :]


def _proj_body(a_ref, b_ref, bias_ref, o_ref):
    acc = lax.dot_general(a_ref[...].astype(jnp.bfloat16),
                          b_ref[...].astype(jnp.bfloat16),
                          (((1,), (0,)), ((), ())),
                          preferred_element_type=jnp.float32)
    o_ref[...] = acc + bias_ref[0:1, :]


def _qkv_body(hs_ref, wq_ref, wk_ref, wv_ref, wg_ref, bq_ref, bk_ref,
              bv_ref, bg_ref, yq_ref, yk_ref, yv_ref, yg_ref):
    j = pl.program_id(0)
    a = hs_ref[...]
    dn = (((1,), (0,)), ((), ()))
    yq_ref[...] = lax.dot_general(
        a, wq_ref[...], dn, preferred_element_type=jnp.float32) + bq_ref[0:1]
    yk_ref[...] = lax.dot_general(
        a, wk_ref[...], dn, preferred_element_type=jnp.float32) + bk_ref[0:1]
    yv_ref[...] = lax.dot_general(
        a, wv_ref[...], dn, preferred_element_type=jnp.float32) + bv_ref[0:1]

    @pl.when(j == 0)
    def _():
        i = pl.program_id(1)
        yg_ref[pl.ds(i * a.shape[0], a.shape[0]), :] = lax.dot_general(
            a, wg_ref[...], dn,
            preferred_element_type=jnp.float32) + bg_ref[0:1]


def _projection(hs, wq, wk, wv, wg_pad, bq, bk, bv, bg_pad):
    BN = 512
    BM = 512
    nj = H // BN
    ni = L // BM
    return pl.pallas_call(
        _qkv_body,
        grid=(nj, ni),
        in_specs=[
            pl.BlockSpec((BM, H), lambda j, i: (i, 0)),
            pl.BlockSpec((H, BN), lambda j, i: (0, j)),
            pl.BlockSpec((H, BN), lambda j, i: (0, j)),
            pl.BlockSpec((H, BN), lambda j, i: (0, j)),
            pl.BlockSpec((H, 128), lambda j, i: (0, 0)),
            pl.BlockSpec((8, BN), lambda j, i: (0, j)),
            pl.BlockSpec((8, BN), lambda j, i: (0, j)),
            pl.BlockSpec((8, BN), lambda j, i: (0, j)),
            pl.BlockSpec((8, 128), lambda j, i: (0, 0)),
        ],
        out_specs=[
            pl.BlockSpec((BM, BN), lambda j, i: (i, j)),
            pl.BlockSpec((BM, BN), lambda j, i: (i, j)),
            pl.BlockSpec((BM, BN), lambda j, i: (i, j)),
            pl.BlockSpec((L, 128), lambda j, i: (0, 0)),
        ],
        out_shape=[
            jax.ShapeDtypeStruct((L, H), jnp.float32),
            jax.ShapeDtypeStruct((L, H), jnp.float32),
            jax.ShapeDtypeStruct((L, H), jnp.float32),
            jax.ShapeDtypeStruct((L, 128), jnp.float32),
        ],
        compiler_params=pltpu.CompilerParams(
            dimension_semantics=("arbitrary", "arbitrary")),
    )(hs, wq, wk, wv, wg_pad,
      jnp.broadcast_to(bq, (8, H)), jnp.broadcast_to(bk, (8, H)),
      jnp.broadcast_to(bv, (8, H)), jnp.broadcast_to(bg_pad, (8, 128)))


def _compress_body(q_ref, k_ref, v_ref, comp_ref, idx_ref, kh_ref, vh_ref):
    h = pl.program_id(0)
    q = q_ref[...]
    k = k_ref[...]
    v = v_ref[...]
    kh_ref[0] = k
    vh_ref[0] = v

    ck = jnp.mean(k.reshape(NB, BS, DH), axis=1)
    cv = jnp.mean(v.reshape(NB, BS, DH), axis=1)

    s = lax.dot_general(q, ck, (((1,), (1,)), ((), ())),
                        preferred_element_type=jnp.float32) * SCALE
    m = jnp.max(s, axis=-1, keepdims=True)
    e = jnp.exp(s - m)
    p = e / jnp.sum(e, axis=-1, keepdims=True)
    comp_ref[...] = lax.dot_general(p, cv, (((1,), (0,)), ((), ())),
                                    preferred_element_type=jnp.float32)

    imp = jnp.sum(p, axis=0, keepdims=True)
    eye = jnp.where(lax.broadcasted_iota(jnp.int32, (NB, NB), 0)
                    == lax.broadcasted_iota(jnp.int32, (NB, NB), 1),
                    jnp.float32(1.0), jnp.float32(0.0))
    imp_c = lax.dot_general(eye, imp, (((1,), (1,)), ((), ())),
                            preferred_element_type=jnp.float32)

    gt = imp > imp_c
    tie = (imp == imp_c) & (lax.broadcasted_iota(jnp.int32, (NB, NB), 1)
                            < lax.broadcasted_iota(jnp.int32, (NB, NB), 0))
    rank = jnp.sum((gt | tie).astype(jnp.float32), axis=1, keepdims=True)
    selm = rank < NSEL

    slot_i = lax.broadcasted_iota(jnp.int32, (NB, NSEL), 1).astype(jnp.float32)
    oh = ((rank == slot_i) & selm).astype(jnp.float32)
    r_i = lax.broadcasted_iota(jnp.int32, (NB, NSEL), 0).astype(jnp.float32)
    blk_row = jnp.sum(r_i * oh, axis=0, keepdims=True)

    pos = lax.broadcasted_iota(jnp.int32, (NSEL * BS, 1), 0)
    slot_of = pos // BS
    oh_pos = (lax.broadcasted_iota(jnp.int32, (NSEL * BS, NSEL), 1)
              == slot_of).astype(jnp.float32)
    blk_of = jnp.sum(oh_pos * blk_row, axis=1, keepdims=True)
    idx_ref[0] = (blk_of.astype(jnp.int32) * BS
                  + (pos - slot_of * BS) + h * L)


def _compress_select(yq, yk, yv):
    return pl.pallas_call(
        _compress_body,
        grid=(NH,),
        in_specs=[
            pl.BlockSpec((L, DH), lambda h: (0, h)),
            pl.BlockSpec((L, DH), lambda h: (0, h)),
            pl.BlockSpec((L, DH), lambda h: (0, h)),
        ],
        out_specs=[
            pl.BlockSpec((L, DH), lambda h: (0, h)),
            pl.BlockSpec((1, NSEL * BS, 1), lambda h: (h, 0, 0)),
            pl.BlockSpec((1, L, DH), lambda h: (h, 0, 0)),
            pl.BlockSpec((1, L, DH), lambda h: (h, 0, 0)),
        ],
        out_shape=[
            jax.ShapeDtypeStruct((L, H), jnp.float32),
            jax.ShapeDtypeStruct((NH, NSEL * BS, 1), jnp.int32),
            jax.ShapeDtypeStruct((NH, L, DH), jnp.float32),
            jax.ShapeDtypeStruct((NH, L, DH), jnp.float32),
        ],
        compiler_params=pltpu.CompilerParams(
            dimension_semantics=("parallel",)),
    )(yq, yk, yv)


NROW = NSEL * BS
NCH = NROW // 128


def _sc_gather_body(ktab, vtab, idx_hbm, out_hbm, idx_v, rows_v, sem):
    c = lax.axis_index("c")
    s = lax.axis_index("s")
    pltpu.sync_copy(idx_hbm.at[s], idx_v)

    @pl.when(c == 0)
    def _():
        cps = [pltpu.async_copy(ktab.at[idx_v.at[j]],
                                rows_v.at[pl.ds(j * 128, 128)], sem)
               for j in range(NCH)]
        for cp in cps:
            cp.wait()

    @pl.when(c == 1)
    def _():
        cps = [pltpu.async_copy(vtab.at[idx_v.at[j]],
                                rows_v.at[pl.ds(j * 128, 128)], sem)
               for j in range(NCH)]
        for cp in cps:
            cp.wait()

    pltpu.sync_copy(rows_v, out_hbm.at[c * NH + s])


def _sc_gather(ktab, vtab, idx3):
    mesh = plsc.VectorSubcoreMesh(core_axis_name="c", subcore_axis_name="s")
    fn = pl.kernel(
        _sc_gather_body,
        out_type=jax.ShapeDtypeStruct((2 * NH, NROW, DH), jnp.float32),
        mesh=mesh,
        scratch_types=[
            pltpu.VMEM((NCH, 128), jnp.int32),
            pltpu.VMEM((NROW, DH), jnp.float32),
            pltpu.SemaphoreType.DMA,
        ],
    )
    return fn(ktab, vtab, idx3)


def _attn_body(q_ref, kc_ref, kp_ref, vc_ref, vp_ref, sk_ref, sv_ref,
               comp_ref, gate_ref, o_ref):
    qi = pl.program_id(1)
    q = q_ref[...].astype(jnp.bfloat16)

    sk = sk_ref[0].astype(jnp.bfloat16)
    sv = sv_ref[0].astype(jnp.bfloat16)
    ss = lax.dot_general(q, sk, (((1,), (1,)), ((), ())),
                         preferred_element_type=jnp.float32) * SCALE
    ms = jnp.max(ss, axis=-1, keepdims=True)
    es = jnp.exp(ss - ms)
    sel_out = lax.dot_general(es.astype(jnp.bfloat16), sv,
                              (((1,), (0,)), ((), ())),
                              preferred_element_type=jnp.float32)
    sel_out = sel_out / jnp.sum(es, axis=-1, keepdims=True)

    i_ = lax.broadcasted_iota(jnp.int32, (QT, QT), 0)
    j_ = lax.broadcasted_iota(jnp.int32, (QT, QT), 1)
    kc = kc_ref[0].astype(jnp.bfloat16)
    vc = vc_ref[0].astype(jnp.bfloat16)
    kp = kp_ref[0].astype(jnp.bfloat16)
    vp = vp_ref[0].astype(jnp.bfloat16)
    sc = lax.dot_general(q, kc, (((1,), (1,)), ((), ())),
                         preferred_element_type=jnp.float32) * SCALE
    sc = jnp.where(i_ >= j_, sc, NEG)
    sp = lax.dot_general(q, kp, (((1,), (1,)), ((), ())),
                         preferred_element_type=jnp.float32) * SCALE
    sp = jnp.where((j_ > i_) & (qi > 0), sp, NEG)
    m = jnp.maximum(jnp.max(sc, axis=-1, keepdims=True),
                    jnp.max(sp, axis=-1, keepdims=True))
    ec = jnp.exp(sc - m)
    ep = jnp.exp(sp - m)
    den = jnp.sum(ec, axis=-1, keepdims=True) + jnp.sum(ep, axis=-1,
                                                        keepdims=True)
    sl_out = (lax.dot_general(ec.astype(jnp.bfloat16), vc,
                              (((1,), (0,)), ((), ())),
                              preferred_element_type=jnp.float32)
              + lax.dot_general(ep.astype(jnp.bfloat16), vp,
                                (((1,), (0,)), ((), ())),
                                preferred_element_type=jnp.float32)) / den

    g = jax.nn.sigmoid(gate_ref[...])
    o_ref[...] = (g[:, 0:1] * comp_ref[...]
                  + g[:, 1:2] * sel_out
                  + g[:, 2:3] * sl_out)


def _attend_combine(yq, khead, vhead, skv, comp, yg):
    return pl.pallas_call(
        _attn_body,
        grid=(NH, NQT),
        in_specs=[
            pl.BlockSpec((QT, DH), lambda h, qi: (qi, h)),
            pl.BlockSpec((1, QT, DH), lambda h, qi: (h, qi, 0)),
            pl.BlockSpec((1, QT, DH),
                         lambda h, qi: (h, jnp.maximum(qi - 1, 0), 0)),
            pl.BlockSpec((1, QT, DH), lambda h, qi: (h, qi, 0)),
            pl.BlockSpec((1, QT, DH),
                         lambda h, qi: (h, jnp.maximum(qi - 1, 0), 0)),
            pl.BlockSpec((1, NROW, DH), lambda h, qi: (h, 0, 0)),
            pl.BlockSpec((1, NROW, DH), lambda h, qi: (NH + h, 0, 0)),
            pl.BlockSpec((QT, DH), lambda h, qi: (qi, h)),
            pl.BlockSpec((QT, 128), lambda h, qi: (qi, 0)),
        ],
        out_specs=pl.BlockSpec((QT, DH), lambda h, qi: (qi, h)),
        out_shape=jax.ShapeDtypeStruct((L, H), jnp.float32),
        compiler_params=pltpu.CompilerParams(
            dimension_semantics=("parallel", "arbitrary")),
    )(yq, khead, khead, vhead, vhead, skv, skv, comp, yg)


def _out_proj(attn, wo, bo):
    return pl.pallas_call(
        _proj_body,
        grid=(H // 512,),
        in_specs=[
            pl.BlockSpec((L, H), lambda j: (0, 0)),
            pl.BlockSpec((H, 512), lambda j: (0, j)),
            pl.BlockSpec((8, 512), lambda j: (0, j)),
        ],
        out_specs=pl.BlockSpec((L, 512), lambda j: (0, j)),
        out_shape=jax.ShapeDtypeStruct((L, H), jnp.float32),
        compiler_params=pltpu.CompilerParams(
            dimension_semantics=("arbitrary",)),
    )(attn, wo, jnp.broadcast_to(bo, (8, H)))


@jax.jit
def kernel(hidden_states, Wq, bq, Wk, bk, Wv, bv, Wo, bo, Wg, bg):
    hs = hidden_states.reshape(L, H)
    wg_pad = jnp.zeros((H, 128), jnp.float32).at[:, :3].set(Wg)
    bg_pad = jnp.zeros((128,), jnp.float32).at[:3].set(bg)

    yq, yk, yv, yg = _projection(hs, Wq, Wk, Wv, wg_pad, bq, bk, bv, bg_pad)
    comp, idx, khead, vhead = _compress_select(yq, yk, yv)
    skv = _sc_gather(khead.reshape(NH * L, DH),
                     vhead.reshape(NH * L, DH),
                     idx.reshape(NH, NCH, 128))
    attn = _attend_combine(yq, khead, vhead, skv, comp, yg)
    out = _out_proj(attn, Wo, bo)
    return out.reshape(B_, L, H)


# speedup vs baseline: 1.2107x; 1.0401x over previous
import functools

import jax
import jax.numpy as jnp
from jax import lax
from jax.experimental import pallas as pl
from jax.experimental.pallas import tpu as pltpu
from jax.experimental.pallas import tpu_sc as plsc

B_, L, H = 1, 2048, 2048
NH, DH = 16, 128
BS = 32
NB = L // BS
NSEL = 16
WIN = 256
QT = 256
NQT = L // QT
SCALE = 1.0 / (DH ** 0.5)
NEG = -1e9


def _proj_body_hi(a_ref, b_ref, bias_ref, o_ref):
    acc = lax.dot_general(a_ref[...], b_ref[...], (((1,), (0,)), ((), ())),
                          preferred_element_type=jnp.float32)
    o_ref[...] = acc + bias_ref[0:1, :]


def _proj_body(a_ref, b_ref, bias_ref, o_ref):
    acc = lax.dot_general(a_ref[...].astype(jnp.bfloat16),
                          b_ref[...].astype(jnp.bfloat16),
                          (((1,), (0,)), ((), ())),
                          preferred_element_type=jnp.float32)
    o_ref[...] = acc + bias_ref[0:1, :]


def _qkv_body(hs_ref, wq_ref, wk_ref, wv_ref, wg_ref, bq_ref, bk_ref,
              bv_ref, bg_ref, yq_ref, yk_ref, yv_ref, yg_ref):
    j = pl.program_id(0)
    a = hs_ref[...]
    dn = (((1,), (0,)), ((), ()))
    yq_ref[...] = lax.dot_general(
        a, wq_ref[...], dn, preferred_element_type=jnp.float32) + bq_ref[0:1]
    yk_ref[...] = lax.dot_general(
        a, wk_ref[...], dn, preferred_element_type=jnp.float32) + bk_ref[0:1]
    yv_ref[...] = lax.dot_general(
        a, wv_ref[...], dn, preferred_element_type=jnp.float32) + bv_ref[0:1]

    @pl.when(j == 0)
    def _():
        i = pl.program_id(1)
        yg_ref[pl.ds(i * a.shape[0], a.shape[0]), :] = jax.nn.sigmoid(
            lax.dot_general(a, wg_ref[...], dn,
                            preferred_element_type=jnp.float32) + bg_ref[0:1])


def _projection(hs, wq, wk, wv, wg_pad, bq, bk, bv, bg_pad):
    BN = 512
    BM = 512
    nj = H // BN
    ni = L // BM
    return pl.pallas_call(
        _qkv_body,
        grid=(nj, ni),
        in_specs=[
            pl.BlockSpec((BM, H), lambda j, i: (i, 0)),
            pl.BlockSpec((H, BN), lambda j, i: (0, j)),
            pl.BlockSpec((H, BN), lambda j, i: (0, j)),
            pl.BlockSpec((H, BN), lambda j, i: (0, j)),
            pl.BlockSpec((H, 128), lambda j, i: (0, 0)),
            pl.BlockSpec((8, BN), lambda j, i: (0, j)),
            pl.BlockSpec((8, BN), lambda j, i: (0, j)),
            pl.BlockSpec((8, BN), lambda j, i: (0, j)),
            pl.BlockSpec((8, 128), lambda j, i: (0, 0)),
        ],
        out_specs=[
            pl.BlockSpec((BM, BN), lambda j, i: (i, j)),
            pl.BlockSpec((BM, BN), lambda j, i: (i, j)),
            pl.BlockSpec((BM, BN), lambda j, i: (i, j)),
            pl.BlockSpec((L, 128), lambda j, i: (0, 0)),
        ],
        out_shape=[
            jax.ShapeDtypeStruct((L, H), jnp.float32),
            jax.ShapeDtypeStruct((L, H), jnp.float32),
            jax.ShapeDtypeStruct((L, H), jnp.float32),
            jax.ShapeDtypeStruct((L, 128), jnp.float32),
        ],
        compiler_params=pltpu.CompilerParams(
            dimension_semantics=("arbitrary", "arbitrary")),
    )(hs, wq, wk, wv, wg_pad,
      jnp.broadcast_to(bq, (8, H)), jnp.broadcast_to(bk, (8, H)),
      jnp.broadcast_to(bv, (8, H)), jnp.broadcast_to(bg_pad, (8, 128)))


def _compress_body(q_ref, k_ref, v_ref, comp_ref, idx_ref, kh_ref, vh_ref):
    h = pl.program_id(0)
    q = q_ref[...]
    k = k_ref[...]
    v = v_ref[...]
    kh_ref[0] = k
    vh_ref[0] = v

    ck = jnp.mean(k.reshape(NB, BS, DH), axis=1)
    cv = jnp.mean(v.reshape(NB, BS, DH), axis=1)

    s = lax.dot_general(q, ck, (((1,), (1,)), ((), ())),
                        preferred_element_type=jnp.float32) * SCALE
    m = jnp.max(s, axis=-1, keepdims=True)
    e = jnp.exp(s - m)
    p = e / jnp.sum(e, axis=-1, keepdims=True)
    comp_ref[...] = lax.dot_general(p, cv, (((1,), (0,)), ((), ())),
                                    preferred_element_type=jnp.float32)

    imp = jnp.sum(p, axis=0, keepdims=True)
    eye = jnp.where(lax.broadcasted_iota(jnp.int32, (NB, NB), 0)
                    == lax.broadcasted_iota(jnp.int32, (NB, NB), 1),
                    jnp.float32(1.0), jnp.float32(0.0))
    imp_c = lax.dot_general(eye, imp, (((1,), (1,)), ((), ())),
                            preferred_element_type=jnp.float32)

    gt = imp > imp_c
    tie = (imp == imp_c) & (lax.broadcasted_iota(jnp.int32, (NB, NB), 1)
                            < lax.broadcasted_iota(jnp.int32, (NB, NB), 0))
    rank = jnp.sum((gt | tie).astype(jnp.float32), axis=1, keepdims=True)
    selm = rank < NSEL

    slot_i = lax.broadcasted_iota(jnp.int32, (NB, NSEL), 1).astype(jnp.float32)
    oh = ((rank == slot_i) & selm).astype(jnp.float32)
    r_i = lax.broadcasted_iota(jnp.int32, (NB, NSEL), 0).astype(jnp.float32)
    blk_row = jnp.sum(r_i * oh, axis=0, keepdims=True)

    pos = lax.broadcasted_iota(jnp.int32, (NSEL * BS, 1), 0)
    slot_of = pos // BS
    oh_pos = (lax.broadcasted_iota(jnp.int32, (NSEL * BS, NSEL), 1)
              == slot_of).astype(jnp.float32)
    blk_of = jnp.sum(oh_pos * blk_row, axis=1, keepdims=True)
    idx_ref[0] = (blk_of.astype(jnp.int32) * BS
                  + (pos - slot_of * BS) + h * L)


def _compress_select(yq, yk, yv):
    return pl.pallas_call(
        _compress_body,
        grid=(NH,),
        in_specs=[
            pl.BlockSpec((L, DH), lambda h: (0, h)),
            pl.BlockSpec((L, DH), lambda h: (0, h)),
            pl.BlockSpec((L, DH), lambda h: (0, h)),
        ],
        out_specs=[
            pl.BlockSpec((L, DH), lambda h: (0, h)),
            pl.BlockSpec((1, NSEL * BS, 1), lambda h: (h, 0, 0)),
            pl.BlockSpec((1, L, DH), lambda h: (h, 0, 0)),
            pl.BlockSpec((1, L, DH), lambda h: (h, 0, 0)),
        ],
        out_shape=[
            jax.ShapeDtypeStruct((L, H), jnp.float32),
            jax.ShapeDtypeStruct((NH, NSEL * BS, 1), jnp.int32),
            jax.ShapeDtypeStruct((NH, L, DH), jnp.float32),
            jax.ShapeDtypeStruct((NH, L, DH), jnp.float32),
        ],
        compiler_params=pltpu.CompilerParams(
            dimension_semantics=("parallel",)),
    )(yq, yk, yv)


NROW = NSEL * BS
NCH = NROW // 128


def _sc_gather_body(ktab, vtab, idx_hbm, out_hbm, idx_v, rows_v, sem):
    c = lax.axis_index("c")
    s = lax.axis_index("s")
    pltpu.sync_copy(idx_hbm.at[s], idx_v)

    @pl.when(c == 0)
    def _():
        cps = [pltpu.async_copy(ktab.at[idx_v.at[j]],
                                rows_v.at[pl.ds(j * 128, 128)], sem)
               for j in range(NCH)]
        for cp in cps:
            cp.wait()

    @pl.when(c == 1)
    def _():
        cps = [pltpu.async_copy(vtab.at[idx_v.at[j]],
                                rows_v.at[pl.ds(j * 128, 128)], sem)
               for j in range(NCH)]
        for cp in cps:
            cp.wait()

    pltpu.sync_copy(rows_v, out_hbm.at[c * NH + s])


def _sc_gather(ktab, vtab, idx3):
    mesh = plsc.VectorSubcoreMesh(core_axis_name="c", subcore_axis_name="s")
    fn = pl.kernel(
        _sc_gather_body,
        out_type=jax.ShapeDtypeStruct((2 * NH, NROW, DH), jnp.float32),
        mesh=mesh,
        scratch_types=[
            pltpu.VMEM((NCH, 128), jnp.int32),
            pltpu.VMEM((NROW, DH), jnp.float32),
            pltpu.SemaphoreType.DMA,
        ],
    )
    return fn(ktab, vtab, idx3)


def _attn_body(q_ref, kc_ref, kp_ref, vc_ref, vp_ref, sk_ref, sv_ref,
               comp_ref, gate_ref, o_ref):
    qi = pl.program_id(1)
    q = (q_ref[...] * SCALE).astype(jnp.bfloat16)

    sk = sk_ref[0].astype(jnp.bfloat16)
    sv = sv_ref[0].astype(jnp.bfloat16)
    ss = lax.dot_general(q, sk, (((1,), (1,)), ((), ())),
                         preferred_element_type=jnp.float32)
    es = jnp.exp(ss)
    sel_out = lax.dot_general(es.astype(jnp.bfloat16), sv,
                              (((1,), (0,)), ((), ())),
                              preferred_element_type=jnp.float32)
    sel_out = sel_out / jnp.sum(es, axis=-1, keepdims=True)

    i_ = lax.broadcasted_iota(jnp.int32, (QT, QT), 0)
    j_ = lax.broadcasted_iota(jnp.int32, (QT, QT), 1)
    kc = kc_ref[0].astype(jnp.bfloat16)
    vc = vc_ref[0].astype(jnp.bfloat16)
    kp = kp_ref[0].astype(jnp.bfloat16)
    vp = vp_ref[0].astype(jnp.bfloat16)
    cmask = (i_ >= j_).astype(jnp.float32)
    pmask = ((j_ > i_) & (qi > 0)).astype(jnp.float32)
    sc = lax.dot_general(q, kc, (((1,), (1,)), ((), ())),
                         preferred_element_type=jnp.float32)
    sp = lax.dot_general(q, kp, (((1,), (1,)), ((), ())),
                         preferred_element_type=jnp.float32)
    ec = jnp.exp(sc) * cmask
    ep = jnp.exp(sp) * pmask
    den = jnp.sum(ec, axis=-1, keepdims=True) + jnp.sum(ep, axis=-1,
                                                        keepdims=True)
    sl_out = (lax.dot_general(ec.astype(jnp.bfloat16), vc,
                              (((1,), (0,)), ((), ())),
                              preferred_element_type=jnp.float32)
              + lax.dot_general(ep.astype(jnp.bfloat16), vp,
                                (((1,), (0,)), ((), ())),
                                preferred_element_type=jnp.float32)) / den

    g = gate_ref[...]
    o_ref[...] = (g[:, 0:1] * comp_ref[...]
                  + g[:, 1:2] * sel_out
                  + g[:, 2:3] * sl_out)


def _attend_combine(yq, khead, vhead, skv, comp, yg):
    return pl.pallas_call(
        _attn_body,
        grid=(NH, NQT),
        in_specs=[
            pl.BlockSpec((QT, DH), lambda h, qi: (qi, h)),
            pl.BlockSpec((1, QT, DH), lambda h, qi: (h, qi, 0)),
            pl.BlockSpec((1, QT, DH),
                         lambda h, qi: (h, jnp.maximum(qi - 1, 0), 0)),
            pl.BlockSpec((1, QT, DH), lambda h, qi: (h, qi, 0)),
            pl.BlockSpec((1, QT, DH),
                         lambda h, qi: (h, jnp.maximum(qi - 1, 0), 0)),
            pl.BlockSpec((1, NROW, DH), lambda h, qi: (h, 0, 0)),
            pl.BlockSpec((1, NROW, DH), lambda h, qi: (NH + h, 0, 0)),
            pl.BlockSpec((QT, DH), lambda h, qi: (qi, h)),
            pl.BlockSpec((QT, 128), lambda h, qi: (qi, 0)),
        ],
        out_specs=pl.BlockSpec((QT, DH), lambda h, qi: (qi, h)),
        out_shape=jax.ShapeDtypeStruct((L, H), jnp.float32),
        compiler_params=pltpu.CompilerParams(
            dimension_semantics=("parallel", "arbitrary")),
    )(yq, khead, khead, vhead, vhead, skv, skv, comp, yg)


def _out_proj(attn, wo, bo):
    return pl.pallas_call(
        _proj_body,
        grid=(H // 512,),
        in_specs=[
            pl.BlockSpec((L, H), lambda j: (0, 0)),
            pl.BlockSpec((H, 512), lambda j: (0, j)),
            pl.BlockSpec((8, 512), lambda j: (0, j)),
        ],
        out_specs=pl.BlockSpec((L, 512), lambda j: (0, j)),
        out_shape=jax.ShapeDtypeStruct((L, H), jnp.float32),
        compiler_params=pltpu.CompilerParams(
            dimension_semantics=("arbitrary",)),
    )(attn, wo, jnp.broadcast_to(bo, (8, H)))


@jax.jit
def kernel(hidden_states, Wq, bq, Wk, bk, Wv, bv, Wo, bo, Wg, bg):
    hs = hidden_states.reshape(L, H)
    wg_pad = jnp.zeros((H, 128), jnp.float32).at[:, :3].set(Wg)
    bg_pad = jnp.zeros((128,), jnp.float32).at[:3].set(bg)

    yq, yk, yv, yg = _projection(hs, Wq, Wk, Wv, wg_pad, bq, bk, bv, bg_pad)
    comp, idx, khead, vhead = _compress_select(yq, yk, yv)
    skv = _sc_gather(khead.reshape(NH * L, DH),
                     vhead.reshape(NH * L, DH),
                     idx.reshape(NH, NCH, 128))
    attn = _attend_combine(yq, khead, vhead, skv, comp, yg)
    out = _out_proj(attn, Wo, bo)
    return out.reshape(B_, L, H)


# speedup vs baseline: 1.2572x; 1.0384x over previous
import functools

import jax
import jax.numpy as jnp
from jax import lax
from jax.experimental import pallas as pl
from jax.experimental.pallas import tpu as pltpu
from jax.experimental.pallas import tpu_sc as plsc

B_, L, H = 1, 2048, 2048
NH, DH = 16, 128
BS = 32
NB = L // BS
NSEL = 16
WIN = 256
QT = 256
NQT = L // QT
SCALE = 1.0 / (DH ** 0.5)
NEG = -1e9


def _proj_body_hi(a_ref, b_ref, bias_ref, o_ref):
    acc = lax.dot_general(a_ref[...], b_ref[...], (((1,), (0,)), ((), ())),
                          preferred_element_type=jnp.float32)
    o_ref[...] = acc + bias_ref[0:1, :]


def _proj_body(a_ref, b_ref, bias_ref, o_ref):
    acc = lax.dot_general(a_ref[...],
                          b_ref[...].astype(jnp.bfloat16),
                          (((1,), (0,)), ((), ())),
                          preferred_element_type=jnp.float32)
    o_ref[...] = acc + bias_ref[0:1, :]


def _qkv_body(hs_ref, wq_ref, wk_ref, wv_ref, wg_ref, bq_ref, bk_ref,
              bv_ref, bg_ref, yq_ref, yk_ref, yv_ref, yg_ref):
    j = pl.program_id(0)
    a = hs_ref[...]
    dn = (((1,), (0,)), ((), ()))
    yq_ref[...] = lax.dot_general(
        a, wq_ref[...], dn, preferred_element_type=jnp.float32) + bq_ref[0:1]
    yk_ref[...] = lax.dot_general(
        a, wk_ref[...], dn, preferred_element_type=jnp.float32) + bk_ref[0:1]
    yv_ref[...] = lax.dot_general(
        a, wv_ref[...], dn, preferred_element_type=jnp.float32) + bv_ref[0:1]

    @pl.when(j == 0)
    def _():
        i = pl.program_id(1)
        yg_ref[pl.ds(i * a.shape[0], a.shape[0]), :] = jax.nn.sigmoid(
            lax.dot_general(a, wg_ref[...], dn,
                            preferred_element_type=jnp.float32)
            + bg_ref[0:1]).astype(jnp.bfloat16)


def _projection(hs, wq, wk, wv, wg_pad, bq, bk, bv, bg_pad):
    BN = 512
    BM = 512
    nj = H // BN
    ni = L // BM
    return pl.pallas_call(
        _qkv_body,
        grid=(nj, ni),
        in_specs=[
            pl.BlockSpec((BM, H), lambda j, i: (i, 0)),
            pl.BlockSpec((H, BN), lambda j, i: (0, j)),
            pl.BlockSpec((H, BN), lambda j, i: (0, j)),
            pl.BlockSpec((H, BN), lambda j, i: (0, j)),
            pl.BlockSpec((H, 128), lambda j, i: (0, 0)),
            pl.BlockSpec((8, BN), lambda j, i: (0, j)),
            pl.BlockSpec((8, BN), lambda j, i: (0, j)),
            pl.BlockSpec((8, BN), lambda j, i: (0, j)),
            pl.BlockSpec((8, 128), lambda j, i: (0, 0)),
        ],
        out_specs=[
            pl.BlockSpec((BM, BN), lambda j, i: (i, j)),
            pl.BlockSpec((BM, BN), lambda j, i: (i, j)),
            pl.BlockSpec((BM, BN), lambda j, i: (i, j)),
            pl.BlockSpec((L, 128), lambda j, i: (0, 0)),
        ],
        out_shape=[
            jax.ShapeDtypeStruct((L, H), jnp.float32),
            jax.ShapeDtypeStruct((L, H), jnp.float32),
            jax.ShapeDtypeStruct((L, H), jnp.float32),
            jax.ShapeDtypeStruct((L, 128), jnp.bfloat16),
        ],
        compiler_params=pltpu.CompilerParams(
            dimension_semantics=("arbitrary", "arbitrary")),
    )(hs, wq, wk, wv, wg_pad,
      jnp.broadcast_to(bq, (8, H)), jnp.broadcast_to(bk, (8, H)),
      jnp.broadcast_to(bv, (8, H)), jnp.broadcast_to(bg_pad, (8, 128)))


def _compress_body(q_ref, k_ref, v_ref, comp_ref, idx_ref, kh_ref, vh_ref):
    h = pl.program_id(0)
    q = q_ref[...]
    k = k_ref[...]
    v = v_ref[...]
    kh_ref[0] = k
    vh_ref[0] = v

    ck = jnp.mean(k.reshape(NB, BS, DH), axis=1)
    cv = jnp.mean(v.reshape(NB, BS, DH), axis=1)

    s = lax.dot_general(q, ck, (((1,), (1,)), ((), ())),
                        preferred_element_type=jnp.float32) * SCALE
    m = jnp.max(s, axis=-1, keepdims=True)
    e = jnp.exp(s - m)
    p = e / jnp.sum(e, axis=-1, keepdims=True)
    comp_ref[...] = lax.dot_general(
        p, cv, (((1,), (0,)), ((), ())),
        preferred_element_type=jnp.float32).astype(jnp.bfloat16)

    imp = jnp.sum(p, axis=0, keepdims=True)
    eye = jnp.where(lax.broadcasted_iota(jnp.int32, (NB, NB), 0)
                    == lax.broadcasted_iota(jnp.int32, (NB, NB), 1),
                    jnp.float32(1.0), jnp.float32(0.0))
    imp_c = lax.dot_general(eye, imp, (((1,), (1,)), ((), ())),
                            preferred_element_type=jnp.float32)

    gt = imp > imp_c
    tie = (imp == imp_c) & (lax.broadcasted_iota(jnp.int32, (NB, NB), 1)
                            < lax.broadcasted_iota(jnp.int32, (NB, NB), 0))
    rank = jnp.sum((gt | tie).astype(jnp.float32), axis=1, keepdims=True)
    selm = rank < NSEL

    slot_i = lax.broadcasted_iota(jnp.int32, (NB, NSEL), 1).astype(jnp.float32)
    oh = ((rank == slot_i) & selm).astype(jnp.float32)
    r_i = lax.broadcasted_iota(jnp.int32, (NB, NSEL), 0).astype(jnp.float32)
    blk_row = jnp.sum(r_i * oh, axis=0, keepdims=True)

    pos = lax.broadcasted_iota(jnp.int32, (NSEL * BS, 1), 0)
    slot_of = pos // BS
    oh_pos = (lax.broadcasted_iota(jnp.int32, (NSEL * BS, NSEL), 1)
              == slot_of).astype(jnp.float32)
    blk_of = jnp.sum(oh_pos * blk_row, axis=1, keepdims=True)
    idx_ref[0] = (blk_of.astype(jnp.int32) * BS
                  + (pos - slot_of * BS) + h * L)


def _compress_select(yq, yk, yv):
    return pl.pallas_call(
        _compress_body,
        grid=(NH,),
        in_specs=[
            pl.BlockSpec((L, DH), lambda h: (0, h)),
            pl.BlockSpec((L, DH), lambda h: (0, h)),
            pl.BlockSpec((L, DH), lambda h: (0, h)),
        ],
        out_specs=[
            pl.BlockSpec((L, DH), lambda h: (0, h)),
            pl.BlockSpec((1, NSEL * BS, 1), lambda h: (h, 0, 0)),
            pl.BlockSpec((1, L, DH), lambda h: (h, 0, 0)),
            pl.BlockSpec((1, L, DH), lambda h: (h, 0, 0)),
        ],
        out_shape=[
            jax.ShapeDtypeStruct((L, H), jnp.bfloat16),
            jax.ShapeDtypeStruct((NH, NSEL * BS, 1), jnp.int32),
            jax.ShapeDtypeStruct((NH, L, DH), jnp.float32),
            jax.ShapeDtypeStruct((NH, L, DH), jnp.float32),
        ],
        compiler_params=pltpu.CompilerParams(
            dimension_semantics=("parallel",)),
    )(yq, yk, yv)


NROW = NSEL * BS
NCH = NROW // 128


def _sc_gather_body(ktab, vtab, idx_hbm, out_hbm, idx_v, rows_v, sem):
    c = lax.axis_index("c")
    s = lax.axis_index("s")
    pltpu.sync_copy(idx_hbm.at[s], idx_v)

    @pl.when(c == 0)
    def _():
        cps = [pltpu.async_copy(ktab.at[idx_v.at[j]],
                                rows_v.at[pl.ds(j * 128, 128)], sem)
               for j in range(NCH)]
        for cp in cps:
            cp.wait()

    @pl.when(c == 1)
    def _():
        cps = [pltpu.async_copy(vtab.at[idx_v.at[j]],
                                rows_v.at[pl.ds(j * 128, 128)], sem)
               for j in range(NCH)]
        for cp in cps:
            cp.wait()

    pltpu.sync_copy(rows_v, out_hbm.at[c * NH + s])


def _sc_gather(ktab, vtab, idx3):
    mesh = plsc.VectorSubcoreMesh(core_axis_name="c", subcore_axis_name="s")
    fn = pl.kernel(
        _sc_gather_body,
        out_type=jax.ShapeDtypeStruct((2 * NH, NROW, DH), jnp.float32),
        mesh=mesh,
        scratch_types=[
            pltpu.VMEM((NCH, 128), jnp.int32),
            pltpu.VMEM((NROW, DH), jnp.float32),
            pltpu.SemaphoreType.DMA,
        ],
    )
    return fn(ktab, vtab, idx3)


def _attn_body(q_ref, k_ref, v_ref, sk_ref, sv_ref,
               comp_ref, gate_ref, o_ref):
    qi = pl.program_id(1)
    q = (q_ref[...] * SCALE).astype(jnp.bfloat16)

    sk = sk_ref[0].astype(jnp.bfloat16)
    sv = sv_ref[0].astype(jnp.bfloat16)
    ss = lax.dot_general(q, sk, (((1,), (1,)), ((), ())),
                         preferred_element_type=jnp.float32)
    es = jnp.exp(ss)
    sel_out = lax.dot_general(es.astype(jnp.bfloat16), sv,
                              (((1,), (0,)), ((), ())),
                              preferred_element_type=jnp.float32)
    sel_out = sel_out / jnp.sum(es, axis=-1, keepdims=True)

    i_ = lax.broadcasted_iota(jnp.int32, (QT, QT), 0)
    j_ = lax.broadcasted_iota(jnp.int32, (QT, QT), 1)
    pstart = jnp.maximum(qi - 1, 0) * QT
    kc = k_ref[0, pl.ds(qi * QT, QT), :].astype(jnp.bfloat16)
    vc = v_ref[0, pl.ds(qi * QT, QT), :].astype(jnp.bfloat16)
    kp = k_ref[0, pl.ds(pstart, QT), :].astype(jnp.bfloat16)
    vp = v_ref[0, pl.ds(pstart, QT), :].astype(jnp.bfloat16)
    cmask = (i_ >= j_).astype(jnp.float32)
    pmask = ((j_ > i_) & (qi > 0)).astype(jnp.float32)
    sc = lax.dot_general(q, kc, (((1,), (1,)), ((), ())),
                         preferred_element_type=jnp.float32)
    sp = lax.dot_general(q, kp, (((1,), (1,)), ((), ())),
                         preferred_element_type=jnp.float32)
    ec = jnp.exp(sc) * cmask
    ep = jnp.exp(sp) * pmask
    den = jnp.sum(ec, axis=-1, keepdims=True) + jnp.sum(ep, axis=-1,
                                                        keepdims=True)
    sl_out = (lax.dot_general(ec.astype(jnp.bfloat16), vc,
                              (((1,), (0,)), ((), ())),
                              preferred_element_type=jnp.float32)
              + lax.dot_general(ep.astype(jnp.bfloat16), vp,
                                (((1,), (0,)), ((), ())),
                                preferred_element_type=jnp.float32)) / den

    g = gate_ref[...].astype(jnp.float32)
    o_ref[...] = (g[:, 0:1] * comp_ref[...].astype(jnp.float32)
                  + g[:, 1:2] * sel_out
                  + g[:, 2:3] * sl_out).astype(jnp.bfloat16)


def _attend_combine(yq, khead, vhead, skv, comp, yg):
    return pl.pallas_call(
        _attn_body,
        grid=(NH, NQT),
        in_specs=[
            pl.BlockSpec((QT, DH), lambda h, qi: (qi, h)),
            pl.BlockSpec((1, L, DH), lambda h, qi: (h, 0, 0)),
            pl.BlockSpec((1, L, DH), lambda h, qi: (h, 0, 0)),
            pl.BlockSpec((1, NROW, DH), lambda h, qi: (h, 0, 0)),
            pl.BlockSpec((1, NROW, DH), lambda h, qi: (NH + h, 0, 0)),
            pl.BlockSpec((QT, DH), lambda h, qi: (qi, h)),
            pl.BlockSpec((QT, 128), lambda h, qi: (qi, 0)),
        ],
        out_specs=pl.BlockSpec((QT, DH), lambda h, qi: (qi, h)),
        out_shape=jax.ShapeDtypeStruct((L, H), jnp.bfloat16),
        compiler_params=pltpu.CompilerParams(
            dimension_semantics=("parallel", "arbitrary")),
    )(yq, khead, vhead, skv, skv, comp, yg)


def _out_proj(attn, wo, bo):
    return pl.pallas_call(
        _proj_body,
        grid=(H // 512,),
        in_specs=[
            pl.BlockSpec((L, H), lambda j: (0, 0)),
            pl.BlockSpec((H, 512), lambda j: (0, j)),
            pl.BlockSpec((8, 512), lambda j: (0, j)),
        ],
        out_specs=pl.BlockSpec((L, 512), lambda j: (0, j)),
        out_shape=jax.ShapeDtypeStruct((L, H), jnp.float32),
        compiler_params=pltpu.CompilerParams(
            dimension_semantics=("arbitrary",)),
    )(attn, wo, jnp.broadcast_to(bo, (8, H)))


@jax.jit
def kernel(hidden_states, Wq, bq, Wk, bk, Wv, bv, Wo, bo, Wg, bg):
    hs = hidden_states.reshape(L, H)
    wg_pad = jnp.zeros((H, 128), jnp.float32).at[:, :3].set(Wg)
    bg_pad = jnp.zeros((128,), jnp.float32).at[:3].set(bg)

    yq, yk, yv, yg = _projection(hs, Wq, Wk, Wv, wg_pad, bq, bk, bv, bg_pad)
    comp, idx, khead, vhead = _compress_select(yq, yk, yv)
    skv = _sc_gather(khead.reshape(NH * L, DH),
                     vhead.reshape(NH * L, DH),
                     idx.reshape(NH, NCH, 128))
    attn = _attend_combine(yq, khead, vhead, skv, comp, yg)
    out = _out_proj(attn, Wo, bo)
    return out.reshape(B_, L, H)
